# Initial kernel scaffold; baseline (speedup 1.0000x reference)
import jax
import jax.numpy as jnp
from jax.experimental import pallas as pl


def kernel(proposals, features):
    raise NotImplementedError("write your pallas kernel here")


# speedup vs baseline: 12.2817x; 12.2817x over previous
import functools

import jax
import jax.numpy as jnp
from jax.experimental import pallas as pl
from jax.experimental.pallas import tpu as pltpu

IMG_H, IMG_W = 800, 800
OUT = 14
P = 8


def _roi_kernel(props_ref, feat_ref, out_ref, w_ref, *, hf, wf):
    feat = feat_ref[0]
    fx = jnp.float32(hf / IMG_H)
    fy = jnp.float32(wf / IMG_W)
    npix = (OUT // 2) * (OUT // 2)
    ncol = 4 * npix

    col = jax.lax.broadcasted_iota(jnp.int32, (1, ncol), 1)
    q, ij = col // npix, col % npix
    ii = 2 * (ij // (OUT // 2)) + q // 2
    jj = 2 * (ij % (OUT // 2)) + q % 2
    ti = ii.astype(jnp.float32) / jnp.float32(OUT - 1)
    tj = jj.astype(jnp.float32) / jnp.float32(OUT - 1)

    r = jax.lax.broadcasted_iota(jnp.int32, (hf * wf, ncol), 0)
    rowx = (r // wf).astype(jnp.float32)
    rowy = (r % wf).astype(jnp.float32)

    one = jnp.float32(1.0)
    for p in range(P):
        px = props_ref[0, p : p + 1, 0:1]
        py = props_ref[0, p : p + 1, 1:2]
        pw = props_ref[0, p : p + 1, 2:3]
        ph = props_ref[0, p : p + 1, 3:4]
        x0 = jnp.floor(px * fx)
        y0 = jnp.floor(py * fy)
        w = jnp.ceil(pw * fx)
        h = jnp.ceil(ph * fy)
        sx = jnp.clip(x0 + ti * (w - one), 0.0, float(hf - 1))
        sy = jnp.clip(y0 + tj * (h - one), 0.0, float(wf - 1))
        xterm = jnp.maximum(one - jnp.abs(rowx - sx), 0.0)
        yterm = jnp.maximum(one - jnp.abs(rowy - sy), 0.0)
        w_ref[:, p * ncol : (p + 1) * ncol] = xterm * yterm

    o = jnp.dot(feat, w_ref[:], preferred_element_type=jnp.float32)
    for p in range(P):
        base = p * ncol
        m0 = jnp.maximum(o[:, base : base + npix],
                         o[:, base + npix : base + 2 * npix])
        m1 = jnp.maximum(o[:, base + 2 * npix : base + 3 * npix],
                         o[:, base + 3 * npix : base + 4 * npix])
        out_ref[0, p] = jnp.maximum(m0, m1)


@jax.jit
def kernel(proposals, features):
    b, c, hf, wf = features.shape
    n = proposals.shape[1]
    feat2 = features.reshape(b, c, hf * wf)
    npix = (OUT // 2) * (OUT // 2)

    out = pl.pallas_call(
        functools.partial(_roi_kernel, hf=hf, wf=wf),
        grid=(b, n // P),
        in_specs=[
            pl.BlockSpec((1, P, 4), lambda bb, nn: (bb, nn, 0)),
            pl.BlockSpec((1, c, hf * wf), lambda bb, nn: (bb, 0, 0)),
        ],
        out_specs=pl.BlockSpec((1, P, c, npix), lambda bb, nn: (bb, nn, 0, 0)),
        out_shape=jax.ShapeDtypeStruct((b, n, c, npix), jnp.float32),
        scratch_shapes=[pltpu.VMEM((hf * wf, P * 4 * npix), jnp.float32)],
        compiler_params=pltpu.CompilerParams(
            dimension_semantics=("parallel", "arbitrary"),
            vmem_limit_bytes=100 * 1024 * 1024,
        ),
    )(proposals, feat2)
    return out.reshape(b, n, c, OUT // 2, OUT // 2)


# speedup vs baseline: 16.6935x; 1.3592x over previous
import functools

import jax
import jax.numpy as jnp
from jax.experimental import pallas as pl
from jax.experimental.pallas import tpu as pltpu

IMG_H, IMG_W = 800, 800
OUT = 14
P = 8
NPIX = (OUT // 2) * (OUT // 2)
NCOLP = 4 * NPIX


def _roi_kernel(props_ref, feat_ref, out_ref, w_ref, *, hf, wf):
    feat = feat_ref[0]
    fx = jnp.float32(hf / IMG_H)
    fy = jnp.float32(wf / IMG_W)
    ncol = P * NCOLP

    col = jax.lax.broadcasted_iota(jnp.int32, (1, ncol), 1)
    pcol = col // NCOLP
    r196 = col % NCOLP
    q, ij = r196 // NPIX, r196 % NPIX
    ii = 2 * (ij // (OUT // 2)) + q // 2
    jj = 2 * (ij % (OUT // 2)) + q % 2
    ti = ii.astype(jnp.float32) / jnp.float32(OUT - 1)
    tj = jj.astype(jnp.float32) / jnp.float32(OUT - 1)

    zero = jnp.zeros((1, 1), jnp.float32)
    x0v, y0v, wv, hv = zero, zero, zero, zero
    for p in range(P):
        sel = pcol == p
        x0v = jnp.where(sel, props_ref[0, p : p + 1, 0:1], x0v)
        y0v = jnp.where(sel, props_ref[0, p : p + 1, 1:2], y0v)
        wv = jnp.where(sel, props_ref[0, p : p + 1, 2:3], wv)
        hv = jnp.where(sel, props_ref[0, p : p + 1, 3:4], hv)
    one = jnp.float32(1.0)
    x0v = jnp.floor(x0v * fx)
    y0v = jnp.floor(y0v * fy)
    wv = jnp.ceil(wv * fx)
    hv = jnp.ceil(hv * fy)
    sx = jnp.clip(x0v + ti * (wv - one), 0.0, float(hf - 1))
    sy = jnp.clip(y0v + tj * (hv - one), 0.0, float(wf - 1))

    r = jax.lax.broadcasted_iota(jnp.int32, (hf * wf, ncol), 0)
    rowx = (r // wf).astype(jnp.float32)
    rowy = (r % wf).astype(jnp.float32)

    bone = jnp.bfloat16(1.0)
    bzero = jnp.bfloat16(0.0)
    xd = jnp.abs(rowx - sx).astype(jnp.bfloat16)
    yd = jnp.abs(rowy - sy).astype(jnp.bfloat16)
    xt = jnp.maximum(bone - xd, bzero)
    yt = jnp.maximum(bone - yd, bzero)
    w_ref[:] = xt * yt

    o = jnp.dot(feat, w_ref[:], preferred_element_type=jnp.float32)
    for p in range(P):
        base = p * NCOLP
        m0 = jnp.maximum(o[:, base : base + NPIX],
                         o[:, base + NPIX : base + 2 * NPIX])
        m1 = jnp.maximum(o[:, base + 2 * NPIX : base + 3 * NPIX],
                         o[:, base + 3 * NPIX : base + 4 * NPIX])
        out_ref[0, p] = jnp.maximum(m0, m1)


@jax.jit
def kernel(proposals, features):
    b, c, hf, wf = features.shape
    n = proposals.shape[1]
    feat2 = features.reshape(b, c, hf * wf).astype(jnp.bfloat16)

    out = pl.pallas_call(
        functools.partial(_roi_kernel, hf=hf, wf=wf),
        grid=(b, n // P),
        in_specs=[
            pl.BlockSpec((1, P, 4), lambda bb, nn: (bb, nn, 0)),
            pl.BlockSpec((1, c, hf * wf), lambda bb, nn: (bb, 0, 0)),
        ],
        out_specs=pl.BlockSpec((1, P, c, NPIX), lambda bb, nn: (bb, nn, 0, 0)),
        out_shape=jax.ShapeDtypeStruct((b, n, c, NPIX), jnp.float32),
        scratch_shapes=[pltpu.VMEM((hf * wf, P * NCOLP), jnp.bfloat16)],
        compiler_params=pltpu.CompilerParams(
            dimension_semantics=("parallel", "arbitrary"),
            vmem_limit_bytes=100 * 1024 * 1024,
        ),
    )(proposals, feat2)
    return out.reshape(b, n, c, OUT // 2, OUT // 2)


# speedup vs baseline: 26.7514x; 1.6025x over previous
import functools

import jax
import jax.numpy as jnp
from jax.experimental import pallas as pl
from jax.experimental.pallas import tpu as pltpu

IMG_H, IMG_W = 800, 800
OUT = 14
P = 8
NPIX = (OUT // 2) * (OUT // 2)
NCOLP = 4 * NPIX


WFP = 64


def _roi_kernel(props_ref, feat_ref, out_ref, w_ref, *, hf, wf):
    feat = feat_ref[0]
    fx = jnp.float32(hf / IMG_H)
    fy = jnp.float32(wf / IMG_W)
    ncol = P * NCOLP

    col = jax.lax.broadcasted_iota(jnp.int32, (1, ncol), 1)
    pcol = col // NCOLP
    r196 = col % NCOLP
    q, ij = r196 // NPIX, r196 % NPIX
    ii = 2 * (ij // (OUT // 2)) + q // 2
    jj = 2 * (ij % (OUT // 2)) + q % 2
    ti = ii.astype(jnp.float32) / jnp.float32(OUT - 1)
    tj = jj.astype(jnp.float32) / jnp.float32(OUT - 1)

    zero = jnp.zeros((1, 1), jnp.float32)
    x0v, y0v, wv, hv = zero, zero, zero, zero
    for p in range(P):
        sel = pcol == p
        x0v = jnp.where(sel, props_ref[0, p : p + 1, 0:1], x0v)
        y0v = jnp.where(sel, props_ref[0, p : p + 1, 1:2], y0v)
        wv = jnp.where(sel, props_ref[0, p : p + 1, 2:3], wv)
        hv = jnp.where(sel, props_ref[0, p : p + 1, 3:4], hv)
    one = jnp.float32(1.0)
    x0v = jnp.floor(x0v * fx)
    y0v = jnp.floor(y0v * fy)
    wv = jnp.ceil(wv * fx)
    hv = jnp.ceil(hv * fy)
    sx = jnp.clip(x0v + ti * (wv - one), 0.0, float(hf - 1))
    sy = jnp.clip(y0v + tj * (hv - one), 0.0, float(wf - 1))

    bone = jnp.bfloat16(1.0)
    bzero = jnp.bfloat16(0.0)
    rowx = jax.lax.broadcasted_iota(jnp.int32, (hf, ncol), 0).astype(jnp.float32)
    rowy = jax.lax.broadcasted_iota(jnp.int32, (WFP, ncol), 0).astype(jnp.float32)
    xd = jnp.abs(rowx - sx).astype(jnp.bfloat16)
    yd = jnp.abs(rowy - sy).astype(jnp.bfloat16)
    xt = jnp.maximum(bone - xd, bzero)
    yt = jnp.maximum(bone - yd, bzero)
    xt_full = jnp.broadcast_to(xt[:, None, :], (hf, WFP, ncol))
    yt_full = jnp.broadcast_to(yt[None, :, :], (hf, WFP, ncol))
    w_ref[:] = (xt_full * yt_full).reshape(hf * WFP, ncol)

    o = jnp.dot(feat, w_ref[:], preferred_element_type=jnp.float32)
    for p in range(P):
        base = p * NCOLP
        m0 = jnp.maximum(o[:, base : base + NPIX],
                         o[:, base + NPIX : base + 2 * NPIX])
        m1 = jnp.maximum(o[:, base + 2 * NPIX : base + 3 * NPIX],
                         o[:, base + 3 * NPIX : base + 4 * NPIX])
        out_ref[0, p] = jnp.maximum(m0, m1)


@jax.jit
def kernel(proposals, features):
    b, c, hf, wf = features.shape
    n = proposals.shape[1]
    featp = jnp.pad(features.astype(jnp.bfloat16),
                    ((0, 0), (0, 0), (0, 0), (0, WFP - wf)))
    feat2 = featp.reshape(b, c, hf * WFP)

    out = pl.pallas_call(
        functools.partial(_roi_kernel, hf=hf, wf=wf),
        grid=(b, n // P),
        in_specs=[
            pl.BlockSpec((1, P, 4), lambda bb, nn: (bb, nn, 0)),
            pl.BlockSpec((1, c, hf * WFP), lambda bb, nn: (bb, 0, 0)),
        ],
        out_specs=pl.BlockSpec((1, P, c, NPIX), lambda bb, nn: (bb, nn, 0, 0)),
        out_shape=jax.ShapeDtypeStruct((b, n, c, NPIX), jnp.float32),
        scratch_shapes=[pltpu.VMEM((hf * WFP, P * NCOLP), jnp.bfloat16)],
        compiler_params=pltpu.CompilerParams(
            dimension_semantics=("parallel", "arbitrary"),
            vmem_limit_bytes=100 * 1024 * 1024,
        ),
    )(proposals, feat2)
    return out.reshape(b, n, c, OUT // 2, OUT // 2)


# speedup vs baseline: 30.9006x; 1.1551x over previous
import functools

import jax
import jax.numpy as jnp
from jax.experimental import pallas as pl
from jax.experimental.pallas import tpu as pltpu

IMG_H, IMG_W = 800, 800
OUT = 14
P = 16
NPIX = (OUT // 2) * (OUT // 2)
NCOLP = 4 * NPIX


WFP = 64


def _roi_kernel(props_ref, feat_ref, out_ref, *, hf, wf):
    feat = feat_ref[0]
    fx = jnp.float32(hf / IMG_H)
    fy = jnp.float32(wf / IMG_W)
    ncol = P * NCOLP

    col = jax.lax.broadcasted_iota(jnp.int32, (1, ncol), 1)
    pcol = col // NCOLP
    r196 = col % NCOLP
    q, ij = r196 // NPIX, r196 % NPIX
    ii = 2 * (ij // (OUT // 2)) + q // 2
    jj = 2 * (ij % (OUT // 2)) + q % 2
    ti = ii.astype(jnp.float32) / jnp.float32(OUT - 1)
    tj = jj.astype(jnp.float32) / jnp.float32(OUT - 1)

    zero = jnp.zeros((1, 1), jnp.float32)
    x0v, y0v, wv, hv = zero, zero, zero, zero
    for p in range(P):
        sel = pcol == p
        x0v = jnp.where(sel, props_ref[0, p : p + 1, 0:1], x0v)
        y0v = jnp.where(sel, props_ref[0, p : p + 1, 1:2], y0v)
        wv = jnp.where(sel, props_ref[0, p : p + 1, 2:3], wv)
        hv = jnp.where(sel, props_ref[0, p : p + 1, 3:4], hv)
    one = jnp.float32(1.0)
    x0v = jnp.floor(x0v * fx)
    y0v = jnp.floor(y0v * fy)
    wv = jnp.ceil(wv * fx)
    hv = jnp.ceil(hv * fy)
    sx = jnp.clip(x0v + ti * (wv - one), 0.0, float(hf - 1))
    sy = jnp.clip(y0v + tj * (hv - one), 0.0, float(wf - 1))

    bone = jnp.bfloat16(1.0)
    bzero = jnp.bfloat16(0.0)
    rowx = jax.lax.broadcasted_iota(jnp.int32, (hf, ncol), 0).astype(jnp.float32)
    rowy = jax.lax.broadcasted_iota(jnp.int32, (WFP, ncol), 0).astype(jnp.float32)
    xd = jnp.abs(rowx - sx).astype(jnp.bfloat16)
    yd = jnp.abs(rowy - sy).astype(jnp.bfloat16)
    xt = jnp.maximum(bone - xd, bzero)
    yt = jnp.maximum(bone - yd, bzero)
    xt_full = jnp.broadcast_to(xt[:, None, :], (hf, WFP, ncol))
    yt_full = jnp.broadcast_to(yt[None, :, :], (hf, WFP, ncol))
    w = (xt_full * yt_full).reshape(hf * WFP, ncol)

    o = jnp.dot(feat, w, preferred_element_type=jnp.float32)
    for p in range(P):
        base = p * NCOLP
        m0 = jnp.maximum(o[:, base : base + NPIX],
                         o[:, base + NPIX : base + 2 * NPIX])
        m1 = jnp.maximum(o[:, base + 2 * NPIX : base + 3 * NPIX],
                         o[:, base + 3 * NPIX : base + 4 * NPIX])
        out_ref[0, p] = jnp.maximum(m0, m1)


@jax.jit
def kernel(proposals, features):
    b, c, hf, wf = features.shape
    n = proposals.shape[1]
    featp = jnp.pad(features.astype(jnp.bfloat16),
                    ((0, 0), (0, 0), (0, 0), (0, WFP - wf)))
    feat2 = featp.reshape(b, c, hf * WFP)

    out = pl.pallas_call(
        functools.partial(_roi_kernel, hf=hf, wf=wf),
        grid=(b, n // P),
        in_specs=[
            pl.BlockSpec((1, P, 4), lambda bb, nn: (bb, nn, 0)),
            pl.BlockSpec((1, c, hf * WFP), lambda bb, nn: (bb, 0, 0)),
        ],
        out_specs=pl.BlockSpec((1, P, c, NPIX), lambda bb, nn: (bb, nn, 0, 0)),
        out_shape=jax.ShapeDtypeStruct((b, n, c, NPIX), jnp.float32),
        compiler_params=pltpu.CompilerParams(
            dimension_semantics=("parallel", "arbitrary"),
            vmem_limit_bytes=100 * 1024 * 1024,
        ),
    )(proposals, feat2)
    return out.reshape(b, n, c, OUT // 2, OUT // 2)
